# baseline (speedup 1.0000x reference)
import functools

import jax
import jax.numpy as jnp
from jax import lax
from jax.experimental import pallas as pl
from jax.experimental.pallas import tpu as pltpu
from jax.experimental.pallas import tpu_sc as plsc

NC = 2
NS = 16
NW = NC * NS


@functools.partial(jax.jit, static_argnums=(2, 3))
def _sc_gather(table, idx, chunk, b_per_w):
    B = idx.shape[0]
    D = table.shape[1]
    n_chunks = b_per_w // chunk
    mesh = plsc.VectorSubcoreMesh(core_axis_name="c", subcore_axis_name="s")

    @functools.partial(
        pl.kernel,
        mesh=mesh,
        out_type=jax.ShapeDtypeStruct((B, D), jnp.float32),
        scratch_types=[
            pltpu.VMEM((chunk,), jnp.int32),
            pltpu.VMEM((chunk, D), jnp.float32),
            pltpu.SemaphoreType.DMA,
        ],
        compiler_params=pltpu.CompilerParams(use_tc_tiling_on_sc=False),
    )
    def k(table_hbm, idx_hbm, out_hbm, idx_v, rows_v, sem):
        wid = lax.axis_index("s") * NC + lax.axis_index("c")
        w_base = wid * b_per_w

        def body(i, carry):
            base = w_base + i * chunk
            pltpu.sync_copy(idx_hbm.at[pl.ds(base, chunk)], idx_v)
            pltpu.async_copy(table_hbm.at[idx_v], rows_v, sem).wait()
            pltpu.sync_copy(rows_v, out_hbm.at[pl.ds(base, chunk)])
            return carry

        lax.fori_loop(0, n_chunks, body, 0)

    return k(table, idx)


def kernel(x, input_embedding):
    B = x.shape[0] * x.shape[1]
    D = input_embedding.shape[1]
    idx = x.reshape(B).astype(jnp.int32)
    b_per_w = B // NW
    out = _sc_gather(input_embedding, idx, 512, b_per_w)
    return out.reshape(x.shape + (D,))


# speedup vs baseline: 1.0396x; 1.0396x over previous
import functools

import jax
import jax.numpy as jnp
from jax import lax
from jax.experimental import pallas as pl
from jax.experimental.pallas import tpu as pltpu
from jax.experimental.pallas import tpu_sc as plsc

NC = 2
NS = 16
NW = NC * NS


@functools.partial(jax.jit, static_argnums=(2, 3))
def _sc_gather(table, idx, chunk, b_per_w):
    B = idx.shape[0]
    D = table.shape[1]
    n_chunks = b_per_w // chunk
    assert n_chunks * chunk == b_per_w and n_chunks % 2 == 0
    pairs = n_chunks // 2
    mesh = plsc.VectorSubcoreMesh(core_axis_name="c", subcore_axis_name="s")

    @functools.partial(
        pl.kernel,
        mesh=mesh,
        out_type=jax.ShapeDtypeStruct((B, D), jnp.float32),
        scratch_types=[
            pltpu.VMEM((b_per_w,), jnp.int32),
            pltpu.VMEM((chunk, D), jnp.float32),
            pltpu.VMEM((chunk, D), jnp.float32),
            pltpu.SemaphoreType.DMA,
            pltpu.SemaphoreType.DMA,
            pltpu.SemaphoreType.DMA,
            pltpu.SemaphoreType.DMA,
        ],
        compiler_params=pltpu.CompilerParams(use_tc_tiling_on_sc=False),
    )
    def k(table_hbm, idx_hbm, out_hbm, idx_v, rows0, rows1, g0, g1, o0, o1):
        wid = lax.axis_index("s") * NC + lax.axis_index("c")
        w_base = wid * b_per_w
        pltpu.sync_copy(idx_hbm.at[pl.ds(w_base, b_per_w)], idx_v)

        def g_start(c, rows, sem):
            pltpu.async_copy(
                table_hbm.at[idx_v.at[pl.ds(c * chunk, chunk)]], rows, sem)

        def g_wait(rows, sem):
            pltpu.make_async_copy(
                table_hbm.at[idx_v.at[pl.ds(0, chunk)]], rows, sem).wait()

        def o_start(c, rows, sem):
            pltpu.async_copy(
                rows, out_hbm.at[pl.ds(w_base + c * chunk, chunk)], sem)

        def o_wait(c, rows, sem):
            pltpu.make_async_copy(
                rows, out_hbm.at[pl.ds(w_base + c * chunk, chunk)], sem).wait()

        g_start(0, rows0, g0)
        g_start(1, rows1, g1)

        def body(j, carry):
            c = 2 * j
            g_wait(rows0, g0)
            o_start(c, rows0, o0)
            g_wait(rows1, g1)
            o_start(c + 1, rows1, o1)
            o_wait(c, rows0, o0)
            g_start(c + 2, rows0, g0)
            o_wait(c + 1, rows1, o1)
            g_start(c + 3, rows1, g1)
            return carry

        lax.fori_loop(0, pairs - 1, body, 0)

        c = n_chunks - 2
        g_wait(rows0, g0)
        o_start(c, rows0, o0)
        g_wait(rows1, g1)
        o_start(c + 1, rows1, o1)
        o_wait(c, rows0, o0)
        o_wait(c + 1, rows1, o1)

    return k(table, idx)


def kernel(x, input_embedding):
    B = x.shape[0] * x.shape[1]
    D = input_embedding.shape[1]
    idx = x.reshape(B).astype(jnp.int32)
    b_per_w = B // NW
    out = _sc_gather(input_embedding, idx, 640, b_per_w)
    return out.reshape(x.shape + (D,))
